# baseline (speedup 1.0000x reference)
import jax
import jax.numpy as jnp
import numpy as np
from jax.experimental import pallas as pl

NUM_QUANTIZERS = 6
NB_CODE = 1024
CODE_DIM = 128
TILE = 1024
CHAINS = 2


def _rvq_kernel(z_ref, cb_ref, p0_ref, p1_ref, p2_ref, p3_ref,
                out_ref, codes_ref):
    sel_iota = jax.lax.broadcasted_iota(jnp.int32, (TILE, NB_CODE), 1)
    zs = [z_ref[pl.ds(c * TILE, TILE), :] for c in range(CHAINS)]
    residual = list(zs)
    quantized = [jnp.zeros_like(z) for z in zs]
    for q in range(NUM_QUANTIZERS):
        cb = cb_ref[q]
        c2 = jnp.sum(cb * cb, axis=1)
        for c in range(CHAINS):
            rr = jnp.sum(residual[c] * residual[c], axis=1, keepdims=True)
            rc = jax.lax.dot_general(
                residual[c], cb,
                dimension_numbers=(((1,), (1,)), ((), ())),
                preferred_element_type=jnp.float32,
            )
            dist = rr - 2.0 * rc + c2[None, :]
            idx = jnp.argmin(dist, axis=1).astype(jnp.int32)
            onehot = (idx[:, None] == sel_iota).astype(jnp.int8)
            dgi = lambda t: jax.lax.dot_general(
                onehot, t,
                dimension_numbers=(((1,), (0,)), ((), ())),
                preferred_element_type=jnp.int32,
            )
            b0 = (dgi(p0_ref[q]) + 128).astype(jnp.uint32)
            b1 = (dgi(p1_ref[q]) + 128).astype(jnp.uint32)
            b2 = (dgi(p2_ref[q]) + 128).astype(jnp.uint32)
            b3 = (dgi(p3_ref[q]) + 128).astype(jnp.uint32)
            word = b0 | (b1 << np.uint32(8)) | (b2 << np.uint32(16)) | (
                b3 << np.uint32(24))
            qv = jax.lax.bitcast_convert_type(word, jnp.float32)
            quantized[c] = quantized[c] + qv
            residual[c] = residual[c] - qv
            codes_ref[q, pl.ds(c * TILE, TILE)] = idx
    for c in range(CHAINS):
        out_ref[pl.ds(c * TILE, TILE), :] = zs[c] + (quantized[c] - zs[c])


def kernel(z, codebooks):
    b, t, d = z.shape
    n_tok = b * t
    flat = z.reshape(n_tok, d)
    step = TILE * CHAINS
    n_tiles = n_tok // step

    bits = jax.lax.bitcast_convert_type(codebooks, jnp.uint32)
    planes = [
        ((jnp.right_shift(bits, np.uint32(8 * k)) & np.uint32(0xFF))
         .astype(jnp.int32) - 128).astype(jnp.int8)
        for k in range(4)
    ]

    cb_spec = pl.BlockSpec((NUM_QUANTIZERS, NB_CODE, d), lambda i: (0, 0, 0))
    out_flat, codes_raw = pl.pallas_call(
        _rvq_kernel,
        grid=(n_tiles,),
        in_specs=[
            pl.BlockSpec((step, d), lambda i: (i, 0)),
            cb_spec, cb_spec, cb_spec, cb_spec, cb_spec,
        ],
        out_specs=[
            pl.BlockSpec((step, d), lambda i: (i, 0)),
            pl.BlockSpec((8, step), lambda i: (0, i)),
        ],
        out_shape=[
            jax.ShapeDtypeStruct((n_tok, d), jnp.float32),
            jax.ShapeDtypeStruct((8, n_tok), jnp.int32),
        ],
    )(flat, codebooks, *planes)

    out = out_flat.reshape(b, t, d)
    codes = codes_raw[:NUM_QUANTIZERS].reshape(NUM_QUANTIZERS, b, t)
    return out, codes


# speedup vs baseline: 1.1685x; 1.1685x over previous
import jax
import jax.numpy as jnp
import numpy as np
from jax.experimental import pallas as pl

NUM_QUANTIZERS = 6
NB_CODE = 1024
CODE_DIM = 128
TILE = 512
CHAINS = 2


def _rvq_kernel(z_ref, cb_ref, p0_ref, p1_ref, p2_ref, p3_ref,
                out_ref, codes_ref):
    sel_iota = jax.lax.broadcasted_iota(jnp.int32, (TILE, NB_CODE), 1)
    zs = [z_ref[pl.ds(c * TILE, TILE), :] for c in range(CHAINS)]
    residual = list(zs)
    quantized = [jnp.zeros_like(z) for z in zs]
    for q in range(NUM_QUANTIZERS):
        cb = cb_ref[q]
        c2 = jnp.sum(cb * cb, axis=1)
        for c in range(CHAINS):
            rr = jnp.sum(residual[c] * residual[c], axis=1, keepdims=True)
            rc = jax.lax.dot_general(
                residual[c], cb,
                dimension_numbers=(((1,), (1,)), ((), ())),
                preferred_element_type=jnp.float32,
            )
            dist = rr - 2.0 * rc + c2[None, :]
            idx = jnp.argmin(dist, axis=1).astype(jnp.int32)
            onehot = (idx[:, None] == sel_iota).astype(jnp.int8)
            dgi = lambda t: jax.lax.dot_general(
                onehot, t,
                dimension_numbers=(((1,), (0,)), ((), ())),
                preferred_element_type=jnp.int32,
            )
            b0 = (dgi(p0_ref[q]) + 128).astype(jnp.uint32)
            b1 = (dgi(p1_ref[q]) + 128).astype(jnp.uint32)
            b2 = (dgi(p2_ref[q]) + 128).astype(jnp.uint32)
            b3 = (dgi(p3_ref[q]) + 128).astype(jnp.uint32)
            word = b0 | (b1 << np.uint32(8)) | (b2 << np.uint32(16)) | (
                b3 << np.uint32(24))
            qv = jax.lax.bitcast_convert_type(word, jnp.float32)
            quantized[c] = quantized[c] + qv
            residual[c] = residual[c] - qv
            codes_ref[q, pl.ds(c * TILE, TILE)] = idx
    for c in range(CHAINS):
        out_ref[pl.ds(c * TILE, TILE), :] = zs[c] + (quantized[c] - zs[c])


def kernel(z, codebooks):
    b, t, d = z.shape
    n_tok = b * t
    flat = z.reshape(n_tok, d)
    step = TILE * CHAINS
    n_tiles = n_tok // step

    bits = jax.lax.bitcast_convert_type(codebooks, jnp.uint32)
    planes = [
        ((jnp.right_shift(bits, np.uint32(8 * k)) & np.uint32(0xFF))
         .astype(jnp.int32) - 128).astype(jnp.int8)
        for k in range(4)
    ]

    cb_spec = pl.BlockSpec((NUM_QUANTIZERS, NB_CODE, d), lambda i: (0, 0, 0))
    out_flat, codes_raw = pl.pallas_call(
        _rvq_kernel,
        grid=(n_tiles,),
        in_specs=[
            pl.BlockSpec((step, d), lambda i: (i, 0)),
            cb_spec, cb_spec, cb_spec, cb_spec, cb_spec,
        ],
        out_specs=[
            pl.BlockSpec((step, d), lambda i: (i, 0)),
            pl.BlockSpec((8, step), lambda i: (0, i)),
        ],
        out_shape=[
            jax.ShapeDtypeStruct((n_tok, d), jnp.float32),
            jax.ShapeDtypeStruct((8, n_tok), jnp.int32),
        ],
    )(flat, codebooks, *planes)

    out = out_flat.reshape(b, t, d)
    codes = codes_raw[:NUM_QUANTIZERS].reshape(NUM_QUANTIZERS, b, t)
    return out, codes


# speedup vs baseline: 1.2628x; 1.0807x over previous
import jax
import jax.numpy as jnp
import numpy as np
from jax.experimental import pallas as pl

NUM_QUANTIZERS = 6
NB_CODE = 1024
CODE_DIM = 128
TILE = 1024
CHAINS = 1


def _rvq_kernel(z_ref, cb_ref, p0_ref, p1_ref, p2_ref, p3_ref,
                out_ref, codes_ref):
    sel_iota = jax.lax.broadcasted_iota(jnp.int32, (TILE, NB_CODE), 1)
    zs = [z_ref[pl.ds(c * TILE, TILE), :] for c in range(CHAINS)]
    residual = list(zs)
    quantized = [jnp.zeros_like(z) for z in zs]
    for q in range(NUM_QUANTIZERS):
        cb = cb_ref[q]
        c2 = jnp.sum(cb * cb, axis=1)
        for c in range(CHAINS):
            rr = jnp.sum(residual[c] * residual[c], axis=1, keepdims=True)
            rc = jax.lax.dot_general(
                residual[c], cb,
                dimension_numbers=(((1,), (1,)), ((), ())),
                preferred_element_type=jnp.float32,
            )
            dist = rr - 2.0 * rc + c2[None, :]
            mval = jnp.min(dist, axis=1, keepdims=True)
            idx = jnp.min(
                jnp.where(dist == mval, sel_iota, NB_CODE), axis=1
            ).astype(jnp.int32)
            onehot = (idx[:, None] == sel_iota).astype(jnp.int8)
            dgi = lambda t: jax.lax.dot_general(
                onehot, t,
                dimension_numbers=(((1,), (0,)), ((), ())),
                preferred_element_type=jnp.int32,
            )
            b0 = (dgi(p0_ref[q]) + 128).astype(jnp.uint32)
            b1 = (dgi(p1_ref[q]) + 128).astype(jnp.uint32)
            b2 = (dgi(p2_ref[q]) + 128).astype(jnp.uint32)
            b3 = (dgi(p3_ref[q]) + 128).astype(jnp.uint32)
            word = b0 | (b1 << np.uint32(8)) | (b2 << np.uint32(16)) | (
                b3 << np.uint32(24))
            qv = jax.lax.bitcast_convert_type(word, jnp.float32)
            quantized[c] = quantized[c] + qv
            residual[c] = residual[c] - qv
            codes_ref[q, pl.ds(c * TILE, TILE)] = idx
    for c in range(CHAINS):
        out_ref[pl.ds(c * TILE, TILE), :] = zs[c] + (quantized[c] - zs[c])


def kernel(z, codebooks):
    b, t, d = z.shape
    n_tok = b * t
    flat = z.reshape(n_tok, d)
    step = TILE * CHAINS
    n_tiles = n_tok // step

    bits = jax.lax.bitcast_convert_type(codebooks, jnp.uint32)
    planes = [
        ((jnp.right_shift(bits, np.uint32(8 * k)) & np.uint32(0xFF))
         .astype(jnp.int32) - 128).astype(jnp.int8)
        for k in range(4)
    ]

    cb_spec = pl.BlockSpec((NUM_QUANTIZERS, NB_CODE, d), lambda i: (0, 0, 0))
    out_flat, codes_raw = pl.pallas_call(
        _rvq_kernel,
        grid=(n_tiles,),
        in_specs=[
            pl.BlockSpec((step, d), lambda i: (i, 0)),
            cb_spec, cb_spec, cb_spec, cb_spec, cb_spec,
        ],
        out_specs=[
            pl.BlockSpec((step, d), lambda i: (i, 0)),
            pl.BlockSpec((8, step), lambda i: (0, i)),
        ],
        out_shape=[
            jax.ShapeDtypeStruct((n_tok, d), jnp.float32),
            jax.ShapeDtypeStruct((8, n_tok), jnp.int32),
        ],
    )(flat, codebooks, *planes)

    out = out_flat.reshape(b, t, d)
    codes = codes_raw[:NUM_QUANTIZERS].reshape(NUM_QUANTIZERS, b, t)
    return out, codes
